# baseline (speedup 1.0000x reference)
import jax
import jax.numpy as jnp
from jax import lax
from jax.experimental import pallas as pl
from jax.experimental.pallas import tpu as pltpu
from jax.experimental.pallas import tpu_sc as plsc

_VOCAB = 100000
_EMB = 128
_N_NEG = 128
_BATCH = 4096
_CTX = 20

_NW = 32
_BW = _BATCH // _NW
_DG = _EMB // 16
_NB = _EMB // 32
_CB = 4
_CCH = _CB * _CTX
_NCH = _BW // _CB

_HIMASK = -65536


def _split_bf16(row_ref, r, k):
    w = row_ref[r, pl.ds(k * 16, 16)]
    e = plsc.bitcast(jnp.left_shift(w, 16), jnp.float32)
    o = plsc.bitcast(jnp.bitwise_and(w, _HIMASK), jnp.float32)
    return e, o


def _sc_scores_body(ctx_flat_hbm, tgt_hbm, neg_hbm, i_emb_hbm, o_emb_hbm,
                    neg_out_hbm, pos_out_hbm,
                    ctx_idx_v, tgt_idx_v, neg_idx_v,
                    cbuf0, cbuf1, tgt_buf, nbuf0, nbuf1,
                    avg_v, scores_v, pos_v,
                    sem_idx, sem_c0, sem_c1, sem_t, sem_n0, sem_n1, sem_out):
    wid = lax.axis_index("s") * 2 + lax.axis_index("c")
    base = wid * _BW

    ci = pltpu.async_copy(
        ctx_flat_hbm.at[pl.ds(base * _CTX, _BW * _CTX)], ctx_idx_v, sem_idx)
    ni = pltpu.async_copy(neg_hbm.at[pl.ds(base, _BW), :], neg_idx_v, sem_idx)
    ti = pltpu.async_copy(tgt_hbm.at[pl.ds(base, _BW)], tgt_idx_v, sem_idx)
    ci.wait()
    ni.wait()
    ti.wait()

    tcp = pltpu.async_copy(o_emb_hbm.at[tgt_idx_v], tgt_buf, sem_t)
    pltpu.make_async_copy(o_emb_hbm.at[neg_idx_v.at[0]], nbuf0, sem_n0).start()
    pltpu.make_async_copy(o_emb_hbm.at[neg_idx_v.at[1]], nbuf1, sem_n1).start()
    pltpu.make_async_copy(
        i_emb_hbm.at[ctx_idx_v.at[pl.ds(0, _CCH)]], cbuf0, sem_c0).start()
    pltpu.make_async_copy(
        i_emb_hbm.at[ctx_idx_v.at[pl.ds(_CCH, _CCH)]], cbuf1, sem_c1).start()

    def _ctx_start(c, buf, sem):
        pltpu.make_async_copy(
            i_emb_hbm.at[ctx_idx_v.at[pl.ds(c * _CCH, _CCH)]], buf, sem).start()

    def _ctx_accum(c, buf):
        for b_loc in range(_CB):
            acc = [None] * _DG
            for j in range(_CTX):
                for k in range(_NB):
                    e, o = _split_bf16(buf, b_loc * _CTX + j, k)
                    if j == 0:
                        acc[2 * k], acc[2 * k + 1] = e, o
                    else:
                        acc[2 * k] += e
                        acc[2 * k + 1] += o
            row = c * _CB + b_loc
            for g in range(_DG):
                avg_v[row, pl.ds(g * 16, 16)] = acc[g]

    def _ctx_pair(p, carry):
        c0 = p * 2
        pltpu.make_async_copy(i_emb_hbm.at[ctx_idx_v.at[pl.ds(0, _CCH)]],
                              cbuf0, sem_c0).wait()
        _ctx_accum(c0, cbuf0)

        @pl.when(p < _NCH // 2 - 1)
        def _start0():
            _ctx_start(c0 + 2, cbuf0, sem_c0)

        pltpu.make_async_copy(i_emb_hbm.at[ctx_idx_v.at[pl.ds(0, _CCH)]],
                              cbuf1, sem_c1).wait()
        _ctx_accum(c0 + 1, cbuf1)

        @pl.when(p < _NCH // 2 - 1)
        def _start1():
            _ctx_start(c0 + 3, cbuf1, sem_c1)

        return carry
    lax.fori_loop(0, _NCH // 2, _ctx_pair, 0)

    tcp.wait()

    inv_ctx = 1.0 / _CTX
    lane = lax.broadcasted_iota(jnp.int32, (16,), 0)
    masks = [lane == l for l in range(16)]

    def _neg_start(b, buf, sem):
        pltpu.make_async_copy(o_emb_hbm.at[neg_idx_v.at[b]], buf, sem).start()

    def _neg_wait(buf, sem):
        pltpu.make_async_copy(o_emb_hbm.at[neg_idx_v.at[0]], buf, sem).wait()

    def _dot(buf, r, a):
        e, o = _split_bf16(buf, r, 0)
        acc = e * a[0]
        acc += o * a[1]
        for k in range(1, _NB):
            e, o = _split_bf16(buf, r, k)
            acc += e * a[2 * k]
            acc += o * a[2 * k + 1]
        return jnp.sum(acc)

    def _row_compute(b, buf, v_pos):
        a = [avg_v[b, pl.ds(g * 16, 16)] * inv_ctx for g in range(_DG)]

        def _per_group(ng, _n):
            v = jnp.zeros((16,), jnp.float32)
            n0 = ng * 16
            for l in range(16):
                v = jnp.where(masks[l], _dot(buf, n0 + l, a), v)
            scores_v[b, pl.ds(n0, 16)] = v
            return _n
        lax.fori_loop(0, _N_NEG // 16, _per_group, 0)

        v_pos = jnp.where(lane == (b % 16), _dot(tgt_buf, b, a), v_pos)

        @pl.when(b % 16 == 15)
        def _flush():
            pos_v[pl.ds(b - 15, 16)] = v_pos

        return v_pos

    def _pair(t, v_pos):
        b0 = t * 2
        _neg_wait(nbuf0, sem_n0)
        v_pos = _row_compute(b0, nbuf0, v_pos)

        @pl.when(t < _BW // 2 - 1)
        def _startn0():
            _neg_start(b0 + 2, nbuf0, sem_n0)

        _neg_wait(nbuf1, sem_n1)
        v_pos = _row_compute(b0 + 1, nbuf1, v_pos)

        @pl.when(t < _BW // 2 - 1)
        def _startn1():
            _neg_start(b0 + 3, nbuf1, sem_n1)

        return v_pos
    lax.fori_loop(0, _BW // 2, _pair, jnp.zeros((16,), jnp.float32))

    pltpu.async_copy(scores_v, neg_out_hbm.at[pl.ds(base, _BW), :], sem_out).wait()
    pltpu.async_copy(pos_v, pos_out_hbm.at[pl.ds(base, _BW)], sem_out).wait()


@jax.jit
def _sc_scores(ctx_flat, tgt, neg, i_emb, o_emb):
    mesh = plsc.VectorSubcoreMesh(core_axis_name="c", subcore_axis_name="s")
    return pl.kernel(
        _sc_scores_body,
        mesh=mesh,
        compiler_params=pltpu.CompilerParams(
            needs_layout_passes=False, use_tc_tiling_on_sc=False),
        out_type=[
            jax.ShapeDtypeStruct((_BATCH, _N_NEG), jnp.float32),
            jax.ShapeDtypeStruct((_BATCH,), jnp.float32),
        ],
        scratch_types=[
            pltpu.VMEM((_BW * _CTX,), jnp.int32),
            pltpu.VMEM((_BW,), jnp.int32),
            pltpu.VMEM((_BW, _N_NEG), jnp.int32),
            pltpu.VMEM((_CCH, _EMB // 2), jnp.int32),
            pltpu.VMEM((_CCH, _EMB // 2), jnp.int32),
            pltpu.VMEM((_BW, _EMB // 2), jnp.int32),
            pltpu.VMEM((_N_NEG, _EMB // 2), jnp.int32),
            pltpu.VMEM((_N_NEG, _EMB // 2), jnp.int32),
            pltpu.VMEM((_BW, _EMB), jnp.float32),
            pltpu.VMEM((_BW, _N_NEG), jnp.float32),
            pltpu.VMEM((_BW,), jnp.float32),
            pltpu.SemaphoreType.DMA,
            pltpu.SemaphoreType.DMA,
            pltpu.SemaphoreType.DMA,
            pltpu.SemaphoreType.DMA,
            pltpu.SemaphoreType.DMA,
            pltpu.SemaphoreType.DMA,
            pltpu.SemaphoreType.DMA,
        ],
    )(ctx_flat, tgt, neg, i_emb, o_emb)


def _loss_body(neg_ref, pos_ref, out_ref):
    def logsig(t):
        return jnp.minimum(t, 0.0) - jnp.log(1.0 + jnp.exp(-jnp.abs(t)))

    total = jnp.sum(logsig(-neg_ref[...])) + jnp.sum(logsig(pos_ref[...]))
    out_ref[0, 0] = -total


@jax.jit
def _tc_loss(neg_dots, pos_dots):
    return pl.pallas_call(
        _loss_body,
        out_shape=jax.ShapeDtypeStruct((1, 1), jnp.float32),
        in_specs=[
            pl.BlockSpec(memory_space=pltpu.VMEM),
            pl.BlockSpec(memory_space=pltpu.VMEM),
        ],
        out_specs=pl.BlockSpec(memory_space=pltpu.SMEM),
    )(neg_dots, pos_dots)


def kernel(target_wids, context_wids, i_embeddings, o_embeddings):
    neg_wids = jax.random.randint(
        jax.random.key(1234), (_BATCH, _N_NEG), 0, _VOCAB - 1, dtype=jnp.int32)
    ctx_flat = context_wids.astype(jnp.int32).reshape(-1)
    tgt = target_wids.astype(jnp.int32)
    i_words = lax.bitcast_convert_type(
        i_embeddings.astype(jnp.bfloat16).reshape(_VOCAB + 1, _EMB // 2, 2),
        jnp.int32)
    o_words = lax.bitcast_convert_type(
        o_embeddings.astype(jnp.bfloat16).reshape(_VOCAB + 1, _EMB // 2, 2),
        jnp.int32)

    neg_dots, pos_dots = _sc_scores(ctx_flat, tgt, neg_wids, i_words, o_words)
    loss = _tc_loss(neg_dots, pos_dots.reshape(_NW, _BW))
    return loss.reshape(())


# speedup vs baseline: 4.9342x; 4.9342x over previous
import jax
import jax.numpy as jnp
from jax import lax
from jax.experimental import pallas as pl
from jax.experimental.pallas import tpu as pltpu
from jax.experimental.pallas import tpu_sc as plsc

_VOCAB = 100000
_EMB = 128
_N_NEG = 128
_BATCH = 4096
_CTX = 20

_NW = 32
_BW = _BATCH // _NW
_DG = _EMB // 16
_CB = 4
_CCH = _CB * _CTX
_NCH = _BW // _CB


def _sc_scores_body(ctx_flat_hbm, tgt_hbm, neg_hbm, i_emb_hbm, o_emb_hbm,
                    neg_out_hbm, pos_out_hbm,
                    ctx_idx_v, tgt_idx_v, neg_idx_v,
                    cbuf0, cbuf1, tgt_buf, nbuf0, nbuf1,
                    avg_v, scores_v, pos_v,
                    sem_idx, sem_c0, sem_c1, sem_t, sem_n0, sem_n1, sem_out):
    wid = lax.axis_index("s") * 2 + lax.axis_index("c")
    base = wid * _BW

    ci = pltpu.async_copy(
        ctx_flat_hbm.at[pl.ds(base * _CTX, _BW * _CTX)], ctx_idx_v, sem_idx)
    ni = pltpu.async_copy(neg_hbm.at[pl.ds(base, _BW), :], neg_idx_v, sem_idx)
    ti = pltpu.async_copy(tgt_hbm.at[pl.ds(base, _BW)], tgt_idx_v, sem_idx)
    ci.wait()
    ni.wait()
    ti.wait()

    tcp = pltpu.async_copy(o_emb_hbm.at[tgt_idx_v], tgt_buf, sem_t)
    pltpu.make_async_copy(o_emb_hbm.at[neg_idx_v.at[0]], nbuf0, sem_n0).start()
    pltpu.make_async_copy(o_emb_hbm.at[neg_idx_v.at[1]], nbuf1, sem_n1).start()
    pltpu.make_async_copy(
        i_emb_hbm.at[ctx_idx_v.at[pl.ds(0, _CCH)]], cbuf0, sem_c0).start()
    pltpu.make_async_copy(
        i_emb_hbm.at[ctx_idx_v.at[pl.ds(_CCH, _CCH)]], cbuf1, sem_c1).start()

    def _ctx_start(c, buf, sem):
        pltpu.make_async_copy(
            i_emb_hbm.at[ctx_idx_v.at[pl.ds(c * _CCH, _CCH)]], buf, sem).start()

    def _ctx_accum(c, buf):
        for b_loc in range(_CB):
            acc = [buf[b_loc * _CTX, pl.ds(g * 16, 16)] for g in range(_DG)]
            for j in range(1, _CTX):
                for g in range(_DG):
                    acc[g] += buf[b_loc * _CTX + j, pl.ds(g * 16, 16)]
            row = c * _CB + b_loc
            for g in range(_DG):
                avg_v[row, pl.ds(g * 16, 16)] = acc[g]

    def _ctx_pair(p, carry):
        c0 = p * 2
        pltpu.make_async_copy(i_emb_hbm.at[ctx_idx_v.at[pl.ds(0, _CCH)]],
                              cbuf0, sem_c0).wait()
        _ctx_accum(c0, cbuf0)

        @pl.when(p < _NCH // 2 - 1)
        def _start0():
            _ctx_start(c0 + 2, cbuf0, sem_c0)

        pltpu.make_async_copy(i_emb_hbm.at[ctx_idx_v.at[pl.ds(0, _CCH)]],
                              cbuf1, sem_c1).wait()
        _ctx_accum(c0 + 1, cbuf1)

        @pl.when(p < _NCH // 2 - 1)
        def _start1():
            _ctx_start(c0 + 3, cbuf1, sem_c1)

        return carry
    lax.fori_loop(0, _NCH // 2, _ctx_pair, 0)

    tcp.wait()

    inv_ctx = 1.0 / _CTX
    lane = lax.broadcasted_iota(jnp.int32, (16,), 0)
    masks = [lane == l for l in range(16)]

    def _neg_start(b, buf, sem):
        pltpu.make_async_copy(o_emb_hbm.at[neg_idx_v.at[b]], buf, sem).start()

    def _neg_wait(buf, sem):
        pltpu.make_async_copy(o_emb_hbm.at[neg_idx_v.at[0]], buf, sem).wait()

    def _row_compute(b, buf, v_pos):
        a = [avg_v[b, pl.ds(g * 16, 16)] * inv_ctx for g in range(_DG)]

        def _per_group(ng, _n):
            v = jnp.zeros((16,), jnp.float32)
            n0 = ng * 16
            for l in range(16):
                n = n0 + l
                acc = buf[n, pl.ds(0, 16)] * a[0]
                for g in range(1, _DG):
                    acc += buf[n, pl.ds(g * 16, 16)] * a[g]
                v = jnp.where(masks[l], jnp.sum(acc), v)
            scores_v[b, pl.ds(n0, 16)] = v
            return _n
        lax.fori_loop(0, _N_NEG // 16, _per_group, 0)

        pacc = tgt_buf[b, pl.ds(0, 16)] * a[0]
        for g in range(1, _DG):
            pacc += tgt_buf[b, pl.ds(g * 16, 16)] * a[g]
        v_pos = jnp.where(lane == (b % 16), jnp.sum(pacc), v_pos)

        @pl.when(b % 16 == 15)
        def _flush():
            pos_v[pl.ds(b - 15, 16)] = v_pos

        return v_pos

    def _pair(t, v_pos):
        b0 = t * 2
        _neg_wait(nbuf0, sem_n0)
        v_pos = _row_compute(b0, nbuf0, v_pos)

        @pl.when(t < _BW // 2 - 1)
        def _startn0():
            _neg_start(b0 + 2, nbuf0, sem_n0)

        _neg_wait(nbuf1, sem_n1)
        v_pos = _row_compute(b0 + 1, nbuf1, v_pos)

        @pl.when(t < _BW // 2 - 1)
        def _startn1():
            _neg_start(b0 + 3, nbuf1, sem_n1)

        return v_pos
    lax.fori_loop(0, _BW // 2, _pair, jnp.zeros((16,), jnp.float32))

    pltpu.async_copy(scores_v, neg_out_hbm.at[pl.ds(base, _BW), :], sem_out).wait()
    pltpu.async_copy(pos_v, pos_out_hbm.at[pl.ds(base, _BW)], sem_out).wait()


@jax.jit
def _sc_scores(ctx_flat, tgt, neg, i_emb, o_emb):
    mesh = plsc.VectorSubcoreMesh(core_axis_name="c", subcore_axis_name="s")
    return pl.kernel(
        _sc_scores_body,
        mesh=mesh,
        compiler_params=pltpu.CompilerParams(
            needs_layout_passes=False,
            skip_device_barrier=True,
            disable_semaphore_checks=True),
        out_type=[
            jax.ShapeDtypeStruct((_BATCH, _N_NEG), jnp.float32),
            jax.ShapeDtypeStruct((_BATCH,), jnp.float32),
        ],
        scratch_types=[
            pltpu.VMEM((_BW * _CTX,), jnp.int32),
            pltpu.VMEM((_BW,), jnp.int32),
            pltpu.VMEM((_BW, _N_NEG), jnp.int32),
            pltpu.VMEM((_CCH, _EMB), jnp.float32),
            pltpu.VMEM((_CCH, _EMB), jnp.float32),
            pltpu.VMEM((_BW, _EMB), jnp.float32),
            pltpu.VMEM((_N_NEG, _EMB), jnp.float32),
            pltpu.VMEM((_N_NEG, _EMB), jnp.float32),
            pltpu.VMEM((_BW, _EMB), jnp.float32),
            pltpu.VMEM((_BW, _N_NEG), jnp.float32),
            pltpu.VMEM((_BW,), jnp.float32),
            pltpu.SemaphoreType.DMA,
            pltpu.SemaphoreType.DMA,
            pltpu.SemaphoreType.DMA,
            pltpu.SemaphoreType.DMA,
            pltpu.SemaphoreType.DMA,
            pltpu.SemaphoreType.DMA,
            pltpu.SemaphoreType.DMA,
        ],
    )(ctx_flat, tgt, neg, i_emb, o_emb)


def _loss_body(neg_ref, pos_ref, out_ref):
    def logsig(t):
        return jnp.minimum(t, 0.0) - jnp.log(1.0 + jnp.exp(-jnp.abs(t)))

    total = jnp.sum(logsig(-neg_ref[...])) + jnp.sum(logsig(pos_ref[...]))
    out_ref[0, 0] = -total


@jax.jit
def _tc_loss(neg_dots, pos_dots):
    return pl.pallas_call(
        _loss_body,
        out_shape=jax.ShapeDtypeStruct((1, 1), jnp.float32),
        in_specs=[
            pl.BlockSpec(memory_space=pltpu.VMEM),
            pl.BlockSpec(memory_space=pltpu.VMEM),
        ],
        out_specs=pl.BlockSpec(memory_space=pltpu.SMEM),
    )(neg_dots, pos_dots)


def kernel(target_wids, context_wids, i_embeddings, o_embeddings):
    neg_wids = jax.random.randint(
        jax.random.key(1234), (_BATCH, _N_NEG), 0, _VOCAB - 1, dtype=jnp.int32)
    ctx_flat = context_wids.astype(jnp.int32).reshape(-1)
    tgt = target_wids.astype(jnp.int32)

    neg_dots, pos_dots = _sc_scores(ctx_flat, tgt, neg_wids,
                                    i_embeddings, o_embeddings)
    loss = _tc_loss(neg_dots, pos_dots.reshape(_NW, _BW))
    return loss.reshape(())


# speedup vs baseline: 5.2311x; 1.0602x over previous
import jax
import jax.numpy as jnp
from jax import lax
from jax.experimental import pallas as pl
from jax.experimental.pallas import tpu as pltpu
from jax.experimental.pallas import tpu_sc as plsc

_VOCAB = 100000
_EMB = 128
_N_NEG = 128
_BATCH = 4096
_CTX = 20

_NW = 32
_BW = _BATCH // _NW
_DG = _EMB // 16
_CB = 4
_CCH = _CB * _CTX
_NCH = _BW // _CB


def _sc_scores_body(ctx_flat_hbm, tgt_hbm, neg_hbm, i_emb_hbm, o_emb_hbm,
                    neg_out_hbm, pos_out_hbm,
                    ctx_idx_v, tgt_idx_v, neg_idx_v,
                    cbuf0, cbuf1, tgt_buf, nbuf0, nbuf1,
                    avg_v, scores_v, pos_v,
                    sem_idx, sem_c0, sem_c1, sem_t, sem_n0, sem_n1, sem_out):
    wid = lax.axis_index("s") * 2 + lax.axis_index("c")
    base = wid * _BW

    ci = pltpu.async_copy(
        ctx_flat_hbm.at[pl.ds(base * _CTX, _BW * _CTX)], ctx_idx_v, sem_idx)
    ni = pltpu.async_copy(neg_hbm.at[pl.ds(base, _BW), :], neg_idx_v, sem_idx)
    ti = pltpu.async_copy(tgt_hbm.at[pl.ds(base, _BW)], tgt_idx_v, sem_idx)
    ci.wait()
    ni.wait()
    ti.wait()

    tcp = pltpu.async_copy(o_emb_hbm.at[tgt_idx_v], tgt_buf, sem_t)
    pltpu.make_async_copy(o_emb_hbm.at[neg_idx_v.at[0]], nbuf0, sem_n0).start()
    pltpu.make_async_copy(o_emb_hbm.at[neg_idx_v.at[1]], nbuf1, sem_n1).start()
    pltpu.make_async_copy(
        i_emb_hbm.at[ctx_idx_v.at[pl.ds(0, _CCH)]], cbuf0, sem_c0).start()
    pltpu.make_async_copy(
        i_emb_hbm.at[ctx_idx_v.at[pl.ds(_CCH, _CCH)]], cbuf1, sem_c1).start()

    def _ctx_start(c, buf, sem):
        pltpu.make_async_copy(
            i_emb_hbm.at[ctx_idx_v.at[pl.ds(c * _CCH, _CCH)]], buf, sem).start()

    def _ctx_accum(c, buf):
        for b_loc in range(_CB):
            acc = [buf[b_loc * _CTX, pl.ds(g * 16, 16)] for g in range(_DG)]
            for j in range(1, _CTX):
                for g in range(_DG):
                    acc[g] += buf[b_loc * _CTX + j, pl.ds(g * 16, 16)]
            row = c * _CB + b_loc
            for g in range(_DG):
                avg_v[row, pl.ds(g * 16, 16)] = acc[g]

    def _ctx_pair(p, carry):
        c0 = p * 2
        pltpu.make_async_copy(i_emb_hbm.at[ctx_idx_v.at[pl.ds(0, _CCH)]],
                              cbuf0, sem_c0).wait()
        _ctx_accum(c0, cbuf0)

        @pl.when(p < _NCH // 2 - 1)
        def _start0():
            _ctx_start(c0 + 2, cbuf0, sem_c0)

        pltpu.make_async_copy(i_emb_hbm.at[ctx_idx_v.at[pl.ds(0, _CCH)]],
                              cbuf1, sem_c1).wait()
        _ctx_accum(c0 + 1, cbuf1)

        @pl.when(p < _NCH // 2 - 1)
        def _start1():
            _ctx_start(c0 + 3, cbuf1, sem_c1)

        return carry
    lax.fori_loop(0, _NCH // 2, _ctx_pair, 0)

    tcp.wait()

    inv_ctx = 1.0 / _CTX
    lane = lax.broadcasted_iota(jnp.int32, (16,), 0)
    masks = [lane == l for l in range(16)]

    def _neg_start(b, buf, sem):
        pltpu.make_async_copy(o_emb_hbm.at[neg_idx_v.at[b]], buf, sem).start()

    def _neg_wait(buf, sem):
        pltpu.make_async_copy(o_emb_hbm.at[neg_idx_v.at[0]], buf, sem).wait()

    def _row_compute(b, buf, v_pos):
        a = [avg_v[b, pl.ds(g * 16, 16)] * inv_ctx for g in range(_DG)]

        def _per_group(ng, _n):
            v = jnp.zeros((16,), jnp.float32)
            n0 = ng * 16
            for l in range(16):
                n = n0 + l
                acc = buf[n, pl.ds(0, 16)] * a[0]
                for g in range(1, _DG):
                    acc += buf[n, pl.ds(g * 16, 16)] * a[g]
                v = jnp.where(masks[l], jnp.sum(acc), v)
            scores_v[b, pl.ds(n0, 16)] = v
            return _n
        lax.fori_loop(0, _N_NEG // 16, _per_group, 0)

        pacc = tgt_buf[b, pl.ds(0, 16)] * a[0]
        for g in range(1, _DG):
            pacc += tgt_buf[b, pl.ds(g * 16, 16)] * a[g]
        v_pos = jnp.where(lane == (b % 16), jnp.sum(pacc), v_pos)

        @pl.when(b % 16 == 15)
        def _flush():
            pos_v[pl.ds(b - 15, 16)] = v_pos

        return v_pos

    def _pair(t, v_pos):
        b0 = t * 2
        _neg_wait(nbuf0, sem_n0)
        v_pos = _row_compute(b0, nbuf0, v_pos)

        @pl.when(t < _BW // 2 - 1)
        def _startn0():
            _neg_start(b0 + 2, nbuf0, sem_n0)

        _neg_wait(nbuf1, sem_n1)
        v_pos = _row_compute(b0 + 1, nbuf1, v_pos)

        @pl.when(t < _BW // 2 - 1)
        def _startn1():
            _neg_start(b0 + 3, nbuf1, sem_n1)

        return v_pos
    lax.fori_loop(0, _BW // 2, _pair, jnp.zeros((16,), jnp.float32))

    pltpu.async_copy(scores_v, neg_out_hbm.at[pl.ds(base, _BW), :], sem_out).wait()
    pltpu.async_copy(pos_v, pos_out_hbm.at[pl.ds(base, _BW)], sem_out).wait()


@jax.jit
def _sc_scores(ctx_flat, tgt, neg, i_emb, o_emb):
    mesh = plsc.VectorSubcoreMesh(core_axis_name="c", subcore_axis_name="s")
    return pl.kernel(
        _sc_scores_body,
        mesh=mesh,
        compiler_params=pltpu.CompilerParams(needs_layout_passes=False),
        out_type=[
            jax.ShapeDtypeStruct((_BATCH, _N_NEG), jnp.float32),
            jax.ShapeDtypeStruct((_BATCH,), jnp.float32),
        ],
        scratch_types=[
            pltpu.VMEM((_BW * _CTX,), jnp.int32),
            pltpu.VMEM((_BW,), jnp.int32),
            pltpu.VMEM((_BW, _N_NEG), jnp.int32),
            pltpu.VMEM((_CCH, _EMB), jnp.float32),
            pltpu.VMEM((_CCH, _EMB), jnp.float32),
            pltpu.VMEM((_BW, _EMB), jnp.float32),
            pltpu.VMEM((_N_NEG, _EMB), jnp.float32),
            pltpu.VMEM((_N_NEG, _EMB), jnp.float32),
            pltpu.VMEM((_BW, _EMB), jnp.float32),
            pltpu.VMEM((_BW, _N_NEG), jnp.float32),
            pltpu.VMEM((_BW,), jnp.float32),
            pltpu.SemaphoreType.DMA,
            pltpu.SemaphoreType.DMA,
            pltpu.SemaphoreType.DMA,
            pltpu.SemaphoreType.DMA,
            pltpu.SemaphoreType.DMA,
            pltpu.SemaphoreType.DMA,
            pltpu.SemaphoreType.DMA,
        ],
    )(ctx_flat, tgt, neg, i_emb, o_emb)


def _loss_body(neg_ref, pos_ref, out_ref):
    def logsig(t):
        return jnp.minimum(t, 0.0) - jnp.log(1.0 + jnp.exp(-jnp.abs(t)))

    total = jnp.sum(logsig(-neg_ref[...])) + jnp.sum(logsig(pos_ref[...]))
    out_ref[0, 0] = -total


@jax.jit
def _tc_loss(neg_dots, pos_dots):
    return pl.pallas_call(
        _loss_body,
        out_shape=jax.ShapeDtypeStruct((1, 1), jnp.float32),
        in_specs=[
            pl.BlockSpec(memory_space=pltpu.VMEM),
            pl.BlockSpec(memory_space=pltpu.VMEM),
        ],
        out_specs=pl.BlockSpec(memory_space=pltpu.SMEM),
    )(neg_dots, pos_dots)


_NEG_CACHE = []


def _neg_wids_const():
    if not _NEG_CACHE:
        with jax.ensure_compile_time_eval():
            _NEG_CACHE.append(jax.random.randint(
                jax.random.key(1234), (_BATCH, _N_NEG), 0, _VOCAB - 1,
                dtype=jnp.int32))
    return _NEG_CACHE[0]


def kernel(target_wids, context_wids, i_embeddings, o_embeddings):
    neg_wids = jnp.asarray(_neg_wids_const())
    ctx_flat = context_wids.astype(jnp.int32).reshape(-1)
    tgt = target_wids.astype(jnp.int32)

    neg_dots, pos_dots = _sc_scores(ctx_flat, tgt, neg_wids,
                                    i_embeddings, o_embeddings)
    loss = _tc_loss(neg_dots, pos_dots.reshape(_NW, _BW))
    return loss.reshape(())
